# baseline (speedup 1.0000x reference)
import functools

import jax
import jax.numpy as jnp
from jax import lax
from jax.experimental import pallas as pl
from jax.experimental.pallas import tpu as pltpu
from jax.experimental.pallas import tpu_sc as plsc

_FEATURE = 128
_DATA = 262144
_TEMP = 0.07
_BATCH = 128

_KBLK = 3072
_NBLK = 4096
_FIXW = 128

_ROWS_PER_SUBCORE = _FEATURE // 32


def _sc_copy_body(bank_hbm, out_hbm):
    wid = lax.axis_index("s") * 2 + lax.axis_index("c")
    r0 = wid * _ROWS_PER_SUBCORE
    pltpu.sync_copy(bank_hbm.at[pl.ds(r0, _ROWS_PER_SUBCORE)],
                    out_hbm.at[pl.ds(r0, _ROWS_PER_SUBCORE)])


def _embed_body(nk, img_ref, w_ref, emb_ref, embT_ref, acc_ref):
    k = pl.program_id(0)

    @pl.when(k == 0)
    def _init():
        acc_ref[...] = jnp.zeros_like(acc_ref)

    acc_ref[...] += jnp.dot(img_ref[...], w_ref[...],
                            preferred_element_type=jnp.float32)

    @pl.when(k == nk - 1)
    def _finish():
        acc = acc_ref[...]
        norm = jnp.sqrt(jnp.sum(acc * acc, axis=1, keepdims=True)) + 1e-12
        emb = acc / norm
        emb_ref[...] = emb
        embT_ref[...] = emb.T


def _logits_body(emb_ref, bank_ref, logits_ref):
    logits_ref[...] = jnp.dot(emb_ref[...], bank_ref[...],
                              preferred_element_type=jnp.float32) * (1.0 / _TEMP)


def _fixup_body(blk_ref, embT_ref, idx_ref, bank_ref, alias_ref, out_ref):
    del alias_ref
    j = pl.program_id(0)
    b = blk_ref[j]
    cols = lax.broadcasted_iota(jnp.int32, (_BATCH, _FIXW), 1) + b * _FIXW
    match = (idx_ref[...] == cols).astype(jnp.float32)
    sel = lax.dot_general(embT_ref[...], match,
                          (((1,), (0,)), ((), ())),
                          preferred_element_type=jnp.float32)
    hit = jnp.max(match, axis=0, keepdims=True)
    out_ref[...] = jnp.where(hit > 0.0, sel, bank_ref[...])


def kernel(images, W, memory_bank, indices):
    feats = images.reshape(_BATCH, -1)
    kdim = feats.shape[1]
    nk = kdim // _KBLK

    bank_copy = pl.kernel(
        _sc_copy_body,
        out_type=jax.ShapeDtypeStruct((_FEATURE, _DATA), jnp.float32),
        mesh=plsc.VectorSubcoreMesh(core_axis_name="c", subcore_axis_name="s"),
    )(memory_bank)

    emb, embT = pl.pallas_call(
        functools.partial(_embed_body, nk),
        grid=(nk,),
        in_specs=[
            pl.BlockSpec((_BATCH, _KBLK), lambda k: (0, k)),
            pl.BlockSpec((_KBLK, _FEATURE), lambda k: (k, 0)),
        ],
        out_specs=[
            pl.BlockSpec((_BATCH, _FEATURE), lambda k: (0, 0)),
            pl.BlockSpec((_FEATURE, _BATCH), lambda k: (0, 0)),
        ],
        out_shape=[
            jax.ShapeDtypeStruct((_BATCH, _FEATURE), jnp.float32),
            jax.ShapeDtypeStruct((_FEATURE, _BATCH), jnp.float32),
        ],
        scratch_shapes=[pltpu.VMEM((_BATCH, _FEATURE), jnp.float32)],
    )(feats, W)

    nj = _DATA // _NBLK
    logits = pl.pallas_call(
        _logits_body,
        grid=(nj,),
        in_specs=[
            pl.BlockSpec((_BATCH, _FEATURE), lambda j: (0, 0)),
            pl.BlockSpec((_FEATURE, _NBLK), lambda j: (0, j)),
        ],
        out_specs=pl.BlockSpec((_BATCH, _NBLK), lambda j: (0, j)),
        out_shape=jax.ShapeDtypeStruct((_BATCH, _DATA), jnp.float32),
    )(emb, memory_bank)

    ar = jnp.arange(_BATCH)
    dup_later = jnp.any(
        (indices[None, :] == indices[:, None]) & (ar[None, :] > ar[:, None]),
        axis=1)
    scatter_idx = jnp.where(dup_later, -1, indices).reshape(_BATCH, 1)
    blk_of = (indices // _FIXW).astype(jnp.int32)

    new_bank = pl.pallas_call(
        _fixup_body,
        grid_spec=pltpu.PrefetchScalarGridSpec(
            num_scalar_prefetch=1,
            grid=(_BATCH,),
            in_specs=[
                pl.BlockSpec((_FEATURE, _BATCH), lambda j, blk: (0, 0)),
                pl.BlockSpec((_BATCH, 1), lambda j, blk: (0, 0)),
                pl.BlockSpec((_FEATURE, _FIXW), lambda j, blk: (0, blk[j])),
                pl.BlockSpec(memory_space=pl.ANY),
            ],
            out_specs=pl.BlockSpec((_FEATURE, _FIXW), lambda j, blk: (0, blk[j])),
        ),
        out_shape=jax.ShapeDtypeStruct((_FEATURE, _DATA), jnp.float32),
        input_output_aliases={4: 0},
    )(blk_of, embT, scatter_idx, memory_bank, bank_copy)

    return (emb, logits, new_bank)


# speedup vs baseline: 10.6030x; 10.6030x over previous
import functools

import jax
import jax.numpy as jnp
from jax import lax
from jax.experimental import pallas as pl
from jax.experimental.pallas import tpu as pltpu
from jax.experimental.pallas import tpu_sc as plsc

_FEATURE = 128
_DATA = 262144
_TEMP = 0.07
_BATCH = 128

_KBLK = 3072
_NBLK = 4096
_FIXW = 128

_ROWS_PER_SUBCORE = _FEATURE // 32
_SC_CH = 16384
_SC_NB = 4
_SC_CPR = _DATA // _SC_CH
_SC_NCH = _ROWS_PER_SUBCORE * _SC_CPR


def _sc_copy_body(bank_hbm, out_hbm, bufs, isems, osems):
    wid = lax.axis_index("s") * 2 + lax.axis_index("c")
    r0 = wid * _ROWS_PER_SUBCORE

    def chunk_slices(i):
        row = r0 + i // _SC_CPR
        col = (i % _SC_CPR) * _SC_CH
        return pl.ds(row, 1), pl.ds(col, _SC_CH)

    in_c = [None] * _SC_NCH
    out_c = [None] * _SC_NCH
    for i in range(_SC_NCH + 1):
        s = i % _SC_NB
        if i < _SC_NCH:
            if i >= _SC_NB:
                out_c[i - _SC_NB].wait()
            rs, cs = chunk_slices(i)
            in_c[i] = pltpu.async_copy(
                bank_hbm.at[rs, cs], bufs.at[pl.ds(s, 1), :], isems.at[s])
        if i >= 1:
            j = i - 1
            sj = j % _SC_NB
            in_c[j].wait()
            rs, cs = chunk_slices(j)
            out_c[j] = pltpu.async_copy(
                bufs.at[pl.ds(sj, 1), :], out_hbm.at[rs, cs], osems.at[sj])
    for j in range(_SC_NCH - _SC_NB, _SC_NCH):
        out_c[j].wait()


def _embed_body(nk, img_ref, w_ref, emb_ref, embT_ref, acc_ref):
    k = pl.program_id(0)

    @pl.when(k == 0)
    def _init():
        acc_ref[...] = jnp.zeros_like(acc_ref)

    acc_ref[...] += jnp.dot(img_ref[...], w_ref[...],
                            preferred_element_type=jnp.float32)

    @pl.when(k == nk - 1)
    def _finish():
        acc = acc_ref[...]
        norm = jnp.sqrt(jnp.sum(acc * acc, axis=1, keepdims=True)) + 1e-12
        emb = acc / norm
        emb_ref[...] = emb
        embT_ref[...] = emb.T


def _logits_body(emb_ref, bank_ref, logits_ref):
    logits_ref[...] = jnp.dot(emb_ref[...], bank_ref[...],
                              preferred_element_type=jnp.float32) * (1.0 / _TEMP)


def _fixup_body(blk_ref, embT_ref, idx_ref, bank_ref, alias_ref, out_ref):
    del alias_ref
    j = pl.program_id(0)
    b = blk_ref[j]
    cols = lax.broadcasted_iota(jnp.int32, (_BATCH, _FIXW), 1) + b * _FIXW
    match = (idx_ref[...] == cols).astype(jnp.float32)
    sel = lax.dot_general(embT_ref[...], match,
                          (((1,), (0,)), ((), ())),
                          preferred_element_type=jnp.float32)
    hit = jnp.max(match, axis=0, keepdims=True)
    out_ref[...] = jnp.where(hit > 0.0, sel, bank_ref[...])


def kernel(images, W, memory_bank, indices):
    feats = images.reshape(_BATCH, -1)
    kdim = feats.shape[1]
    nk = kdim // _KBLK

    bank_copy = pl.kernel(
        _sc_copy_body,
        out_type=jax.ShapeDtypeStruct((_FEATURE, _DATA), jnp.float32),
        mesh=plsc.VectorSubcoreMesh(core_axis_name="c", subcore_axis_name="s"),
        scratch_types=[
            pltpu.VMEM((_SC_NB, _SC_CH), jnp.float32),
            pltpu.SemaphoreType.DMA((_SC_NB,)),
            pltpu.SemaphoreType.DMA((_SC_NB,)),
        ],
    )(memory_bank)

    emb, embT = pl.pallas_call(
        functools.partial(_embed_body, nk),
        grid=(nk,),
        in_specs=[
            pl.BlockSpec((_BATCH, _KBLK), lambda k: (0, k)),
            pl.BlockSpec((_KBLK, _FEATURE), lambda k: (k, 0)),
        ],
        out_specs=[
            pl.BlockSpec((_BATCH, _FEATURE), lambda k: (0, 0)),
            pl.BlockSpec((_FEATURE, _BATCH), lambda k: (0, 0)),
        ],
        out_shape=[
            jax.ShapeDtypeStruct((_BATCH, _FEATURE), jnp.float32),
            jax.ShapeDtypeStruct((_FEATURE, _BATCH), jnp.float32),
        ],
        scratch_shapes=[pltpu.VMEM((_BATCH, _FEATURE), jnp.float32)],
    )(feats, W)

    nj = _DATA // _NBLK
    logits = pl.pallas_call(
        _logits_body,
        grid=(nj,),
        in_specs=[
            pl.BlockSpec((_BATCH, _FEATURE), lambda j: (0, 0)),
            pl.BlockSpec((_FEATURE, _NBLK), lambda j: (0, j)),
        ],
        out_specs=pl.BlockSpec((_BATCH, _NBLK), lambda j: (0, j)),
        out_shape=jax.ShapeDtypeStruct((_BATCH, _DATA), jnp.float32),
    )(emb, memory_bank)

    ar = jnp.arange(_BATCH)
    dup_later = jnp.any(
        (indices[None, :] == indices[:, None]) & (ar[None, :] > ar[:, None]),
        axis=1)
    scatter_idx = jnp.where(dup_later, -1, indices).reshape(_BATCH, 1)
    blk_of = (indices // _FIXW).astype(jnp.int32)

    new_bank = pl.pallas_call(
        _fixup_body,
        grid_spec=pltpu.PrefetchScalarGridSpec(
            num_scalar_prefetch=1,
            grid=(_BATCH,),
            in_specs=[
                pl.BlockSpec((_FEATURE, _BATCH), lambda j, blk: (0, 0)),
                pl.BlockSpec((_BATCH, 1), lambda j, blk: (0, 0)),
                pl.BlockSpec((_FEATURE, _FIXW), lambda j, blk: (0, blk[j])),
                pl.BlockSpec(memory_space=pl.ANY),
            ],
            out_specs=pl.BlockSpec((_FEATURE, _FIXW), lambda j, blk: (0, blk[j])),
        ),
        out_shape=jax.ShapeDtypeStruct((_FEATURE, _DATA), jnp.float32),
        input_output_aliases={4: 0},
    )(blk_of, embT, scatter_idx, memory_bank, bank_copy)

    return (emb, logits, new_bank)


# speedup vs baseline: 14.3581x; 1.3542x over previous
import functools

import jax
import jax.numpy as jnp
from jax import lax
from jax.experimental import pallas as pl
from jax.experimental.pallas import tpu as pltpu

_FEATURE = 128
_DATA = 262144
_TEMP = 0.07
_BATCH = 128
_CH3 = 3
_IMG = 224

_YBLK = 16
_NBLK = 4096


def _embed_body(ng, img_ref, w_ref, emb_ref, embT_ref, acc_ref):
    g = pl.program_id(0)

    @pl.when(g == 0)
    def _init():
        acc_ref[...] = jnp.zeros_like(acc_ref)

    part = jnp.zeros((_BATCH, _FEATURE), jnp.float32)
    for y in range(_YBLK):
        part += jnp.dot(img_ref[:, 0, y, :], w_ref[0, y, :, :],
                        preferred_element_type=jnp.float32)
    acc_ref[...] += part

    @pl.when(g == ng - 1)
    def _finish():
        acc = acc_ref[...]
        norm = jnp.sqrt(jnp.sum(acc * acc, axis=1, keepdims=True)) + 1e-12
        emb = acc / norm
        emb_ref[...] = emb
        embT_ref[...] = emb.T


def _bank_body(emb_ref, embT_ref, idx_ref, bank_ref, logits_ref, nb_ref):
    bank = bank_ref[...]
    logits_ref[...] = jnp.dot(emb_ref[...], bank,
                              preferred_element_type=jnp.float32) * (1.0 / _TEMP)
    j = pl.program_id(0)
    cols = lax.broadcasted_iota(jnp.int32, (_BATCH, _NBLK), 1) + j * _NBLK
    match = (idx_ref[...] == cols).astype(jnp.float32)
    sel = lax.dot_general(embT_ref[...], match,
                          (((1,), (0,)), ((), ())),
                          preferred_element_type=jnp.float32)
    hit = jnp.max(match, axis=0, keepdims=True)
    nb_ref[...] = jnp.where(hit > 0.0, sel, bank)


def kernel(images, W, memory_bank, indices):
    ypg = _IMG // _YBLK
    ng = _CH3 * ypg
    W4 = W.reshape(_CH3, _IMG, _IMG, _FEATURE)
    emb, embT = pl.pallas_call(
        functools.partial(_embed_body, ng),
        grid=(ng,),
        in_specs=[
            pl.BlockSpec((_BATCH, 1, _YBLK, _IMG),
                         lambda g: (0, g // (_IMG // _YBLK), g % (_IMG // _YBLK), 0)),
            pl.BlockSpec((1, _YBLK, _IMG, _FEATURE),
                         lambda g: (g // (_IMG // _YBLK), g % (_IMG // _YBLK), 0, 0)),
        ],
        out_specs=[
            pl.BlockSpec((_BATCH, _FEATURE), lambda g: (0, 0)),
            pl.BlockSpec((_FEATURE, _BATCH), lambda g: (0, 0)),
        ],
        out_shape=[
            jax.ShapeDtypeStruct((_BATCH, _FEATURE), jnp.float32),
            jax.ShapeDtypeStruct((_FEATURE, _BATCH), jnp.float32),
        ],
        scratch_shapes=[pltpu.VMEM((_BATCH, _FEATURE), jnp.float32)],
    )(images, W4)

    ar = jnp.arange(_BATCH)
    dup_later = jnp.any(
        (indices[None, :] == indices[:, None]) & (ar[None, :] > ar[:, None]),
        axis=1)
    scatter_idx = jnp.where(dup_later, -1, indices).reshape(_BATCH, 1)

    nj = _DATA // _NBLK
    logits, new_bank = pl.pallas_call(
        _bank_body,
        grid=(nj,),
        in_specs=[
            pl.BlockSpec((_BATCH, _FEATURE), lambda j: (0, 0)),
            pl.BlockSpec((_FEATURE, _BATCH), lambda j: (0, 0)),
            pl.BlockSpec((_BATCH, 1), lambda j: (0, 0)),
            pl.BlockSpec((_FEATURE, _NBLK), lambda j: (0, j)),
        ],
        out_specs=[
            pl.BlockSpec((_BATCH, _NBLK), lambda j: (0, j)),
            pl.BlockSpec((_FEATURE, _NBLK), lambda j: (0, j)),
        ],
        out_shape=[
            jax.ShapeDtypeStruct((_BATCH, _DATA), jnp.float32),
            jax.ShapeDtypeStruct((_FEATURE, _DATA), jnp.float32),
        ],
    )(emb, embT, scatter_idx, memory_bank)

    return (emb, logits, new_bank)


# speedup vs baseline: 14.3922x; 1.0024x over previous
import functools

import jax
import jax.numpy as jnp
from jax import lax
from jax.experimental import pallas as pl
from jax.experimental.pallas import tpu as pltpu

_FEATURE = 128
_DATA = 262144
_TEMP = 0.07
_BATCH = 128
_CH3 = 3
_IMG = 224

_YBLK = 16
_NBLK = 4096


def _embed_body(ng, img_ref, w_ref, emb_ref, embT_ref, acc_ref):
    g = pl.program_id(0)

    @pl.when(g == 0)
    def _init():
        acc_ref[...] = jnp.zeros_like(acc_ref)

    part = jnp.zeros((_BATCH, _FEATURE), jnp.float32)
    for y in range(_YBLK):
        part += jnp.dot(img_ref[:, 0, y, :], w_ref[y * _IMG:(y + 1) * _IMG, :],
                        preferred_element_type=jnp.float32)
    acc_ref[...] += part

    @pl.when(g == ng - 1)
    def _finish():
        acc = acc_ref[...]
        norm = jnp.sqrt(jnp.sum(acc * acc, axis=1, keepdims=True)) + 1e-12
        emb = acc / norm
        emb_ref[...] = emb
        embT_ref[...] = emb.T


def _bank_body(emb_ref, embT_ref, idx_ref, bank_ref, logits_ref, nb_ref):
    bank = bank_ref[...]
    logits_ref[...] = jnp.dot(emb_ref[...], bank,
                              preferred_element_type=jnp.float32) * (1.0 / _TEMP)
    j = pl.program_id(0)
    cols = lax.broadcasted_iota(jnp.int32, (_BATCH, _NBLK), 1) + j * _NBLK
    match = (idx_ref[...] == cols).astype(jnp.float32)
    sel = lax.dot_general(embT_ref[...], match,
                          (((1,), (0,)), ((), ())),
                          preferred_element_type=jnp.float32)
    hit = jnp.max(match, axis=0, keepdims=True)
    nb_ref[...] = jnp.where(hit > 0.0, sel, bank)


def kernel(images, W, memory_bank, indices):
    ypg = _IMG // _YBLK
    ng = _CH3 * ypg
    emb, embT = pl.pallas_call(
        functools.partial(_embed_body, ng),
        grid=(ng,),
        in_specs=[
            pl.BlockSpec((_BATCH, 1, _YBLK, _IMG),
                         lambda g: (0, g // (_IMG // _YBLK), g % (_IMG // _YBLK), 0)),
            pl.BlockSpec((_YBLK * _IMG, _FEATURE), lambda g: (g, 0)),
        ],
        out_specs=[
            pl.BlockSpec((_BATCH, _FEATURE), lambda g: (0, 0)),
            pl.BlockSpec((_FEATURE, _BATCH), lambda g: (0, 0)),
        ],
        out_shape=[
            jax.ShapeDtypeStruct((_BATCH, _FEATURE), jnp.float32),
            jax.ShapeDtypeStruct((_FEATURE, _BATCH), jnp.float32),
        ],
        scratch_shapes=[pltpu.VMEM((_BATCH, _FEATURE), jnp.float32)],
    )(images, W)

    ar = jnp.arange(_BATCH)
    dup_later = jnp.any(
        (indices[None, :] == indices[:, None]) & (ar[None, :] > ar[:, None]),
        axis=1)
    scatter_idx = jnp.where(dup_later, -1, indices).reshape(_BATCH, 1)

    nj = _DATA // _NBLK
    logits, new_bank = pl.pallas_call(
        _bank_body,
        grid=(nj,),
        in_specs=[
            pl.BlockSpec((_BATCH, _FEATURE), lambda j: (0, 0)),
            pl.BlockSpec((_FEATURE, _BATCH), lambda j: (0, 0)),
            pl.BlockSpec((_BATCH, 1), lambda j: (0, 0)),
            pl.BlockSpec((_FEATURE, _NBLK), lambda j: (0, j)),
        ],
        out_specs=[
            pl.BlockSpec((_BATCH, _NBLK), lambda j: (0, j)),
            pl.BlockSpec((_FEATURE, _NBLK), lambda j: (0, j)),
        ],
        out_shape=[
            jax.ShapeDtypeStruct((_BATCH, _DATA), jnp.float32),
            jax.ShapeDtypeStruct((_FEATURE, _DATA), jnp.float32),
        ],
    )(emb, embT, scatter_idx, memory_bank)

    return (emb, logits, new_bank)


# speedup vs baseline: 20.1779x; 1.4020x over previous
import functools

import jax
import jax.numpy as jnp
from jax import lax
from jax.experimental import pallas as pl
from jax.experimental.pallas import tpu as pltpu

_FEATURE = 128
_DATA = 262144
_TEMP = 0.07
_BATCH = 128
_CH3 = 3
_IMG = 224

_KBLK = 3072
_NBLK = 4096


def _embed_body(ng, img_ref, w_ref, emb_ref, embT_ref, acc_ref):
    g = pl.program_id(0)

    @pl.when(g == 0)
    def _init():
        acc_ref[...] = jnp.zeros_like(acc_ref)

    acc_ref[...] += lax.dot_general(img_ref[...], w_ref[...],
                                    (((0,), (0,)), ((), ())),
                                    preferred_element_type=jnp.float32)

    @pl.when(g == ng - 1)
    def _finish():
        acc = acc_ref[...]
        norm = jnp.sqrt(jnp.sum(acc * acc, axis=1, keepdims=True)) + 1e-12
        emb = acc / norm
        emb_ref[...] = emb
        embT_ref[...] = emb.T


def _bank_body(emb_ref, embT_ref, idx_ref, bank_ref, logits_ref, nb_ref):
    bank = bank_ref[...]
    logits_ref[...] = jnp.dot(emb_ref[...], bank,
                              preferred_element_type=jnp.float32) * (1.0 / _TEMP)
    j = pl.program_id(0)
    cols = lax.broadcasted_iota(jnp.int32, (_BATCH, _NBLK), 1) + j * _NBLK
    match = (idx_ref[...] == cols).astype(jnp.float32)
    sel = lax.dot_general(embT_ref[...], match,
                          (((1,), (0,)), ((), ())),
                          preferred_element_type=jnp.float32)
    hit = jnp.max(match, axis=0, keepdims=True)
    nb_ref[...] = jnp.where(hit > 0.0, sel, bank)


def kernel(images, W, memory_bank, indices):
    kdim = _CH3 * _IMG * _IMG
    imgT = jnp.transpose(images, (1, 2, 3, 0)).reshape(kdim, _BATCH)
    ng = kdim // _KBLK
    emb, embT = pl.pallas_call(
        functools.partial(_embed_body, ng),
        grid=(ng,),
        in_specs=[
            pl.BlockSpec((_KBLK, _BATCH), lambda g: (g, 0)),
            pl.BlockSpec((_KBLK, _FEATURE), lambda g: (g, 0)),
        ],
        out_specs=[
            pl.BlockSpec((_BATCH, _FEATURE), lambda g: (0, 0)),
            pl.BlockSpec((_FEATURE, _BATCH), lambda g: (0, 0)),
        ],
        out_shape=[
            jax.ShapeDtypeStruct((_BATCH, _FEATURE), jnp.float32),
            jax.ShapeDtypeStruct((_FEATURE, _BATCH), jnp.float32),
        ],
        scratch_shapes=[pltpu.VMEM((_BATCH, _FEATURE), jnp.float32)],
    )(imgT, W)

    ar = jnp.arange(_BATCH)
    dup_later = jnp.any(
        (indices[None, :] == indices[:, None]) & (ar[None, :] > ar[:, None]),
        axis=1)
    scatter_idx = jnp.where(dup_later, -1, indices).reshape(_BATCH, 1)

    nj = _DATA // _NBLK
    logits, new_bank = pl.pallas_call(
        _bank_body,
        grid=(nj,),
        in_specs=[
            pl.BlockSpec((_BATCH, _FEATURE), lambda j: (0, 0)),
            pl.BlockSpec((_FEATURE, _BATCH), lambda j: (0, 0)),
            pl.BlockSpec((_BATCH, 1), lambda j: (0, 0)),
            pl.BlockSpec((_FEATURE, _NBLK), lambda j: (0, j)),
        ],
        out_specs=[
            pl.BlockSpec((_BATCH, _NBLK), lambda j: (0, j)),
            pl.BlockSpec((_FEATURE, _NBLK), lambda j: (0, j)),
        ],
        out_shape=[
            jax.ShapeDtypeStruct((_BATCH, _DATA), jnp.float32),
            jax.ShapeDtypeStruct((_FEATURE, _DATA), jnp.float32),
        ],
    )(emb, embT, scatter_idx, memory_bank)

    return (emb, logits, new_bank)


# speedup vs baseline: 23.0131x; 1.1405x over previous
import functools

import jax
import jax.numpy as jnp
from jax import lax
from jax.experimental import pallas as pl
from jax.experimental.pallas import tpu as pltpu

_FEATURE = 128
_DATA = 262144
_TEMP = 0.07
_BATCH = 128
_CH3 = 3
_IMG = 224

_KBLK = 6272
_NBLK = 8192


def _embed_body(ng, img_ref, w_ref, emb_ref, embT_ref, acc_ref):
    g = pl.program_id(0)

    @pl.when(g == 0)
    def _init():
        acc_ref[...] = jnp.zeros_like(acc_ref)

    acc_ref[...] += lax.dot_general(img_ref[...], w_ref[...],
                                    (((0,), (0,)), ((), ())),
                                    preferred_element_type=jnp.float32)

    @pl.when(g == ng - 1)
    def _finish():
        acc = acc_ref[...]
        norm = jnp.sqrt(jnp.sum(acc * acc, axis=1, keepdims=True)) + 1e-12
        emb = acc / norm
        emb_ref[...] = emb
        embT_ref[...] = emb.T


def _bank_body(emb_ref, embT_ref, idx_ref, bank_ref, logits_ref, nb_ref):
    bank = bank_ref[...]
    logits_ref[...] = jnp.dot(emb_ref[...], bank,
                              preferred_element_type=jnp.float32) * (1.0 / _TEMP)
    j = pl.program_id(0)
    cols = lax.broadcasted_iota(jnp.int32, (_BATCH, _NBLK), 1) + j * _NBLK
    match = (idx_ref[...] == cols).astype(jnp.float32)
    sel = lax.dot_general(embT_ref[...], match,
                          (((1,), (0,)), ((), ())),
                          preferred_element_type=jnp.float32)
    hit = jnp.max(match, axis=0, keepdims=True)
    nb_ref[...] = jnp.where(hit > 0.0, sel, bank)


def kernel(images, W, memory_bank, indices):
    kdim = _CH3 * _IMG * _IMG
    imgT = jnp.transpose(images, (1, 2, 3, 0)).reshape(kdim, _BATCH)
    ng = kdim // _KBLK
    emb, embT = pl.pallas_call(
        functools.partial(_embed_body, ng),
        grid=(ng,),
        in_specs=[
            pl.BlockSpec((_KBLK, _BATCH), lambda g: (g, 0)),
            pl.BlockSpec((_KBLK, _FEATURE), lambda g: (g, 0)),
        ],
        out_specs=[
            pl.BlockSpec((_BATCH, _FEATURE), lambda g: (0, 0)),
            pl.BlockSpec((_FEATURE, _BATCH), lambda g: (0, 0)),
        ],
        out_shape=[
            jax.ShapeDtypeStruct((_BATCH, _FEATURE), jnp.float32),
            jax.ShapeDtypeStruct((_FEATURE, _BATCH), jnp.float32),
        ],
        scratch_shapes=[pltpu.VMEM((_BATCH, _FEATURE), jnp.float32)],
    )(imgT, W)

    ar = jnp.arange(_BATCH)
    dup_later = jnp.any(
        (indices[None, :] == indices[:, None]) & (ar[None, :] > ar[:, None]),
        axis=1)
    scatter_idx = jnp.where(dup_later, -1, indices).reshape(_BATCH, 1)

    nj = _DATA // _NBLK
    logits, new_bank = pl.pallas_call(
        _bank_body,
        grid=(nj,),
        in_specs=[
            pl.BlockSpec((_BATCH, _FEATURE), lambda j: (0, 0)),
            pl.BlockSpec((_FEATURE, _BATCH), lambda j: (0, 0)),
            pl.BlockSpec((_BATCH, 1), lambda j: (0, 0)),
            pl.BlockSpec((_FEATURE, _NBLK), lambda j: (0, j)),
        ],
        out_specs=[
            pl.BlockSpec((_BATCH, _NBLK), lambda j: (0, j)),
            pl.BlockSpec((_FEATURE, _NBLK), lambda j: (0, j)),
        ],
        out_shape=[
            jax.ShapeDtypeStruct((_BATCH, _DATA), jnp.float32),
            jax.ShapeDtypeStruct((_FEATURE, _DATA), jnp.float32),
        ],
    )(emb, embT, scatter_idx, memory_bank)

    return (emb, logits, new_bank)


# speedup vs baseline: 23.3835x; 1.0161x over previous
import functools

import jax
import jax.numpy as jnp
from jax import lax
from jax.experimental import pallas as pl
from jax.experimental.pallas import tpu as pltpu

_FEATURE = 128
_DATA = 262144
_TEMP = 0.07
_BATCH = 128
_CH3 = 3
_IMG = 224

_KBLK = 12544
_NBLK = 16384


def _embed_body(ng, img_ref, w_ref, emb_ref, embT_ref, acc_ref):
    g = pl.program_id(0)

    @pl.when(g == 0)
    def _init():
        acc_ref[...] = jnp.zeros_like(acc_ref)

    acc_ref[...] += lax.dot_general(img_ref[...], w_ref[...],
                                    (((0,), (0,)), ((), ())),
                                    preferred_element_type=jnp.float32)

    @pl.when(g == ng - 1)
    def _finish():
        acc = acc_ref[...]
        norm = jnp.sqrt(jnp.sum(acc * acc, axis=1, keepdims=True)) + 1e-12
        emb = acc / norm
        emb_ref[...] = emb
        embT_ref[...] = emb.T


def _bank_body(emb_ref, embT_ref, idx_ref, bank_ref, logits_ref, nb_ref):
    bank = bank_ref[...]
    logits_ref[...] = jnp.dot(emb_ref[...], bank,
                              preferred_element_type=jnp.float32) * (1.0 / _TEMP)
    j = pl.program_id(0)
    cols = lax.broadcasted_iota(jnp.int32, (_BATCH, _NBLK), 1) + j * _NBLK
    match = (idx_ref[...] == cols).astype(jnp.float32)
    sel = lax.dot_general(embT_ref[...], match,
                          (((1,), (0,)), ((), ())),
                          preferred_element_type=jnp.float32)
    hit = jnp.max(match, axis=0, keepdims=True)
    nb_ref[...] = jnp.where(hit > 0.0, sel, bank)


def kernel(images, W, memory_bank, indices):
    kdim = _CH3 * _IMG * _IMG
    imgT = jnp.transpose(images, (1, 2, 3, 0)).reshape(kdim, _BATCH)
    ng = kdim // _KBLK
    emb, embT = pl.pallas_call(
        functools.partial(_embed_body, ng),
        grid=(ng,),
        in_specs=[
            pl.BlockSpec((_KBLK, _BATCH), lambda g: (g, 0)),
            pl.BlockSpec((_KBLK, _FEATURE), lambda g: (g, 0)),
        ],
        out_specs=[
            pl.BlockSpec((_BATCH, _FEATURE), lambda g: (0, 0)),
            pl.BlockSpec((_FEATURE, _BATCH), lambda g: (0, 0)),
        ],
        out_shape=[
            jax.ShapeDtypeStruct((_BATCH, _FEATURE), jnp.float32),
            jax.ShapeDtypeStruct((_FEATURE, _BATCH), jnp.float32),
        ],
        scratch_shapes=[pltpu.VMEM((_BATCH, _FEATURE), jnp.float32)],
    )(imgT, W)

    ar = jnp.arange(_BATCH)
    dup_later = jnp.any(
        (indices[None, :] == indices[:, None]) & (ar[None, :] > ar[:, None]),
        axis=1)
    scatter_idx = jnp.where(dup_later, -1, indices).reshape(_BATCH, 1)

    nj = _DATA // _NBLK
    logits, new_bank = pl.pallas_call(
        _bank_body,
        grid=(nj,),
        in_specs=[
            pl.BlockSpec((_BATCH, _FEATURE), lambda j: (0, 0)),
            pl.BlockSpec((_FEATURE, _BATCH), lambda j: (0, 0)),
            pl.BlockSpec((_BATCH, 1), lambda j: (0, 0)),
            pl.BlockSpec((_FEATURE, _NBLK), lambda j: (0, j)),
        ],
        out_specs=[
            pl.BlockSpec((_BATCH, _NBLK), lambda j: (0, j)),
            pl.BlockSpec((_FEATURE, _NBLK), lambda j: (0, j)),
        ],
        out_shape=[
            jax.ShapeDtypeStruct((_BATCH, _DATA), jnp.float32),
            jax.ShapeDtypeStruct((_FEATURE, _DATA), jnp.float32),
        ],
    )(emb, embT, scatter_idx, memory_bank)

    return (emb, logits, new_bank)
